# baseline (speedup 1.0000x reference)
import functools

import jax
import jax.numpy as jnp
from jax import lax
from jax.experimental import pallas as pl
from jax.experimental.pallas import tpu as pltpu
from jax.experimental.pallas import tpu_sc as plsc

_VOCAB = 100000
_DMODEL = 768
_CTX = 2048
_BATCH = 4
_SEQ = 2048

_NC = 2
_NS = 16
_NW = _NC * _NS
_L = 16

_P = _SEQ // _NW
_C = _P // 2
_NCHUNK = _BATCH * _P // _C
_NBUF = 3


def _embed_body(x_hbm, wword_hbm, wpos_hbm, out_hbm,
                idx_all, pos_v, rows0, rows1, rows2,
                isem, psem, gsem0, gsem1, gsem2, wsem0, wsem1, wsem2):
    wid = lax.axis_index("s") * _NC + lax.axis_index("c")
    pos_base = wid * _P

    rows = [rows0, rows1, rows2]
    gsem = [gsem0, gsem1, gsem2]
    wsem = [wsem0, wsem1, wsem2]

    icps = [
        pltpu.async_copy(x_hbm.at[b, pl.ds(pos_base, _P)],
                         idx_all.at[pl.ds(b * _P, _P)], isem)
        for b in range(_BATCH)
    ]
    pcp = pltpu.async_copy(wpos_hbm.at[pl.ds(pos_base, _P)], pos_v, psem)

    def off(k):
        b, h = divmod(k, 2)
        return b * _SEQ + pos_base + h * _C

    def gather(k):
        return pltpu.async_copy(wword_hbm.at[idx_all.at[pl.ds(k * _C, _C)]],
                                rows[k % _NBUF], gsem[k % _NBUF])

    icps[0].wait()
    gcp = [None] * _NCHUNK
    wcp = [None] * _NCHUNK
    gcp[0] = gather(0)
    for icp in icps[1:]:
        icp.wait()
    gcp[1] = gather(1)

    for k in range(_NCHUNK):
        if k + 2 < _NCHUNK:
            if k >= 1:
                wcp[k - 1].wait()
            gcp[k + 2] = gather(k + 2)
        gcp[k].wait()
        if k == 0:
            pcp.wait()

        buf = rows[k % _NBUF]
        pbase = (k % 2) * _C

        @plsc.parallel_loop(0, _C, step=1, unroll=2)
        def _add(r):
            for c in range(_DMODEL // _L):
                sl = pl.ds(c * _L, _L)
                plsc.addupdate(buf.at[r, sl], pos_v[pbase + r, sl])

        wcp[k] = pltpu.async_copy(buf, out_hbm.at[pl.ds(off(k), _C)],
                                  wsem[k % _NBUF])
    wcp[_NCHUNK - 2].wait()
    wcp[_NCHUNK - 1].wait()


@jax.jit
def _embed(x, W_word, W_pos):
    mesh = plsc.VectorSubcoreMesh(core_axis_name="c", subcore_axis_name="s")
    k = functools.partial(
        pl.kernel,
        mesh=mesh,
        out_type=jax.ShapeDtypeStruct((_BATCH * _SEQ, _DMODEL), jnp.float32),
        scratch_types=[
            pltpu.VMEM((_BATCH * _P,), jnp.int32),
            pltpu.VMEM((_P, _DMODEL), jnp.float32),
            pltpu.VMEM((_C, _DMODEL), jnp.float32),
            pltpu.VMEM((_C, _DMODEL), jnp.float32),
            pltpu.VMEM((_C, _DMODEL), jnp.float32),
            pltpu.SemaphoreType.DMA,
            pltpu.SemaphoreType.DMA,
            pltpu.SemaphoreType.DMA,
            pltpu.SemaphoreType.DMA,
            pltpu.SemaphoreType.DMA,
            pltpu.SemaphoreType.DMA,
            pltpu.SemaphoreType.DMA,
            pltpu.SemaphoreType.DMA,
        ],
    )(_embed_body)
    return k(x, W_word, W_pos)


def kernel(x, W_word, W_pos):
    batch, seq = x.shape
    out = _embed(x.astype(jnp.int32), W_word, W_pos)
    return out.reshape(batch, seq, _DMODEL)


# speedup vs baseline: 1.1510x; 1.1510x over previous
import functools

import jax
import jax.numpy as jnp
from jax import lax
from jax.experimental import pallas as pl
from jax.experimental.pallas import tpu as pltpu
from jax.experimental.pallas import tpu_sc as plsc

_VOCAB = 100000
_DMODEL = 768
_CTX = 2048
_BATCH = 4
_SEQ = 2048

_NC = 2
_NS = 16
_NW = _NC * _NS
_L = 16

_P = _SEQ // _NW
_C = _P // 2


def _embed_body(x_hbm, wword_hbm, wpos_hbm, out_hbm,
                idx_all, pos_v, rows0, rows1,
                isem, psem, gsem0, gsem1, wsem0, wsem1):
    wid = lax.axis_index("s") * _NC + lax.axis_index("c")
    pos_base = wid * _P

    icp0 = pltpu.async_copy(x_hbm.at[0, pl.ds(pos_base, _P)],
                            idx_all.at[pl.ds(0, _P)], isem)
    icps = [
        pltpu.async_copy(x_hbm.at[b, pl.ds(pos_base, _P)],
                         idx_all.at[pl.ds(b * _P, _P)], isem)
        for b in range(1, _BATCH)
    ]
    pcp = pltpu.async_copy(wpos_hbm.at[pl.ds(pos_base, _P)], pos_v, psem)

    def gather(k, rowbuf, sem):
        return pltpu.async_copy(wword_hbm.at[idx_all.at[pl.ds(k * _C, _C)]],
                                rowbuf, sem)

    def drain_gather(rowbuf, sem):
        pltpu.make_async_copy(wword_hbm.at[pl.ds(0, _C)], rowbuf, sem).wait()

    def drain_write(rowbuf, sem):
        pltpu.make_async_copy(rowbuf, out_hbm.at[pl.ds(0, _C)], sem).wait()

    def add_pos(rowbuf, half):
        pbase = half * _C

        @plsc.parallel_loop(0, _C, step=1, unroll=1)
        def _add(r):
            for c in range(_DMODEL // _L):
                sl = pl.ds(c * _L, _L)
                plsc.addupdate(rowbuf.at[r, sl], pos_v[pbase + r, sl])

    icp0.wait()
    gather(0, rows0, gsem0)
    for icp in icps:
        icp.wait()

    def body(b, carry):
        out_off = b * _SEQ + pos_base

        @pl.when(b > 0)
        def _():
            drain_write(rows1, wsem1)
        gather(2 * b + 1, rows1, gsem1)
        drain_gather(rows0, gsem0)

        @pl.when(b == 0)
        def _():
            pltpu.make_async_copy(wpos_hbm.at[pl.ds(0, _P)], pos_v,
                                  psem).wait()

        add_pos(rows0, 0)
        pltpu.async_copy(rows0, out_hbm.at[pl.ds(out_off, _C)], wsem0)

        @pl.when(b < _BATCH - 1)
        def _():
            drain_write(rows0, wsem0)
            gather(2 * b + 2, rows0, gsem0)

        drain_gather(rows1, gsem1)
        add_pos(rows1, 1)
        pltpu.async_copy(rows1, out_hbm.at[pl.ds(out_off + _C, _C)], wsem1)
        return carry

    lax.fori_loop(0, _BATCH, body, 0)
    drain_write(rows0, wsem0)
    drain_write(rows1, wsem1)


@jax.jit
def _embed(x, W_word, W_pos):
    mesh = plsc.VectorSubcoreMesh(core_axis_name="c", subcore_axis_name="s")
    k = functools.partial(
        pl.kernel,
        mesh=mesh,
        out_type=jax.ShapeDtypeStruct((_BATCH * _SEQ, _DMODEL), jnp.float32),
        scratch_types=[
            pltpu.VMEM((_BATCH * _P,), jnp.int32),
            pltpu.VMEM((_P, _DMODEL), jnp.float32),
            pltpu.VMEM((_C, _DMODEL), jnp.float32),
            pltpu.VMEM((_C, _DMODEL), jnp.float32),
            pltpu.SemaphoreType.DMA,
            pltpu.SemaphoreType.DMA,
            pltpu.SemaphoreType.DMA,
            pltpu.SemaphoreType.DMA,
            pltpu.SemaphoreType.DMA,
            pltpu.SemaphoreType.DMA,
        ],
    )(_embed_body)
    return k(x, W_word, W_pos)


def kernel(x, W_word, W_pos):
    batch, seq = x.shape
    out = _embed(x.astype(jnp.int32), W_word, W_pos)
    return out.reshape(batch, seq, _DMODEL)
